# baseline (speedup 1.0000x reference)
import functools

import jax
import jax.numpy as jnp
from jax import lax
from jax.experimental import pallas as pl
from jax.experimental.pallas import tpu as pltpu
from jax.experimental.pallas import tpu_sc as plsc

_R, _C = 128, 100000
_BR = 8
_NBLK = _R // _BR

_NC, _NS = 2, 16
_NW = _NC * _NS
_RPW = _R // _NW
_CHUNK = 20000
_NCHUNK = _C // _CHUNK
_VECS = _CHUNK // 16

_GATHER_DN = lax.GatherDimensionNumbers(
    offset_dims=(), collapsed_slice_dims=(0,), start_index_map=(0,))


def _shuffle(x, perm):
    return lax.gather(x, perm[:, None], _GATHER_DN, slice_sizes=(1,),
                      mode=lax.GatherScatterMode.PROMISE_IN_BOUNDS)


def _sc_body(out_hbm, logit_hbm, lsel_hbm, buf0, buf1, idx_v, gat_v,
             sem0, sem1, gsem):
    cid = lax.axis_index("c")
    sid = lax.axis_index("s")
    wid = sid * _NC + cid
    lane = lax.broadcasted_iota(jnp.int32, (16,), 0)
    idx_acc = jnp.zeros((16,), jnp.int32)

    bufs = (buf0, buf1)
    sems = (sem0, sem1)

    for r in range(_RPW):
        row = (wid * _RPW + r) * _C
        copies = [None] * _NCHUNK
        copies[0] = pltpu.async_copy(
            out_hbm.at[pl.ds(row, _CHUNK)], bufs[0], sems[0])
        runmax = jnp.full((16,), -jnp.inf, jnp.float32)
        runidx = jnp.zeros((16,), jnp.int32)
        for c in range(_NCHUNK):
            if c + 1 < _NCHUNK:
                copies[c + 1] = pltpu.async_copy(
                    out_hbm.at[pl.ds(row + (c + 1) * _CHUNK, _CHUNK)],
                    bufs[(c + 1) % 2], sems[(c + 1) % 2])
            copies[c].wait()
            buf = bufs[c % 2]
            pos0 = lane + c * _CHUNK

            def body(i, carry, buf=buf):
                rmax, ridx, pos = carry
                v = buf[pl.ds(i * 16, 16)]
                upd = v > rmax
                rmax = jnp.where(upd, v, rmax)
                ridx = jnp.where(upd, pos, ridx)
                return rmax, ridx, pos + 16

            runmax, runidx, _ = lax.fori_loop(
                0, _VECS, body, (runmax, runidx, pos0))
        val, idx = runmax, runidx
        for s in (1, 2, 4, 8):
            perm = lane ^ s
            oval = _shuffle(val, perm)
            oidx = _shuffle(idx, perm)
            take = (oval > val) | ((oval == val) & (oidx < idx))
            val = jnp.where(take, oval, val)
            idx = jnp.where(take, oidx, idx)
        idx_acc = jnp.where(lane == r, row + idx, idx_acc)

    idx_v[...] = idx_acc
    pltpu.async_copy(logit_hbm.at[idx_v], gat_v, gsem).wait()
    pltpu.sync_copy(gat_v, lsel_hbm.at[wid])


_sc_argmax_gather = functools.partial(
    pl.kernel,
    out_type=jax.ShapeDtypeStruct((_NW, 16), jnp.float32),
    mesh=plsc.VectorSubcoreMesh(core_axis_name="c", subcore_axis_name="s"),
    scratch_types=[
        pltpu.VMEM((_CHUNK,), jnp.float32),
        pltpu.VMEM((_CHUNK,), jnp.float32),
        pltpu.VMEM((16,), jnp.int32),
        pltpu.VMEM((16,), jnp.float32),
        pltpu.SemaphoreType.DMA,
        pltpu.SemaphoreType.DMA,
        pltpu.SemaphoreType.DMA,
    ],
)(_sc_body)


def _tc_sums_body(l_ref, ent_ref, logs_ref):
    l = l_ref[...]
    s = jnp.sum(l, axis=1, keepdims=True)
    sll = jnp.sum(l * jnp.log(l), axis=1, keepdims=True)
    logs = jnp.log(s)
    ent_ref[...] = jnp.broadcast_to(logs - sll / s, (_BR, 128))
    logs_ref[...] = jnp.broadcast_to(logs, (_BR, 128))


def _combine_body(lsel_ref, logs_ref, alp_ref):
    alp_ref[...] = jnp.log(lsel_ref[...]) - logs_ref[...]


def kernel(logits, outputs):
    lsel = _sc_argmax_gather(outputs.reshape(-1), logits.reshape(-1))
    ent, logs = pl.pallas_call(
        _tc_sums_body,
        grid=(_NBLK,),
        in_specs=[pl.BlockSpec((_BR, _C), lambda i: (i, 0))],
        out_specs=[
            pl.BlockSpec((_BR, 128), lambda i: (i, 0)),
            pl.BlockSpec((_BR, 128), lambda i: (i, 0)),
        ],
        out_shape=[
            jax.ShapeDtypeStruct((_R, 128), jnp.float32),
            jax.ShapeDtypeStruct((_R, 128), jnp.float32),
        ],
    )(logits)
    lsel_row = lsel[:, :_RPW].reshape(1, _R)
    logs_row = logs[:, 0].reshape(1, _R)
    alp = pl.pallas_call(
        _combine_body,
        out_shape=jax.ShapeDtypeStruct((1, _R), jnp.float32),
    )(lsel_row, logs_row)
    return (alp.reshape(_R), ent[:, 0])


# speedup vs baseline: 1.8349x; 1.8349x over previous
import functools

import jax
import jax.numpy as jnp
from jax import lax
from jax.experimental import pallas as pl
from jax.experimental.pallas import tpu as pltpu
from jax.experimental.pallas import tpu_sc as plsc

_R, _C = 128, 100000
_BR = 8
_NBLK = _R // _BR

_CHT = 13
_CHW = _CHT * 128
_NCH = 60
_NCHW = _NCH // 2
_TAILC = _NCH * _CHW
_TAILW = _C - _TAILC
_NEG = -3.4e38

_GATHER_DN = lax.GatherDimensionNumbers(
    offset_dims=(), collapsed_slice_dims=(0,), start_index_map=(0,))


def _shuffle(x, perm):
    return lax.gather(x, perm[:, None], _GATHER_DN, slice_sizes=(1,),
                      mode=lax.GatherScatterMode.PROMISE_IN_BOUNDS)


def _update(v, p, rmax, ridx):
    upd = v > rmax
    return jnp.where(upd, v, rmax), jnp.where(upd, p, ridx)


def _sc_body(out_hbm, logit_hbm, val_hbm, idx_hbm, lsel_hbm,
             buf0, buf1, tbuf, gbuf, valv, idxv, lselv, sem0, sem1, gsem):
    cid = lax.axis_index("c")
    sid = lax.axis_index("s")
    wid = cid * 16 + sid
    row0 = sid * 8
    lane = lax.broadcasted_iota(jnp.int32, (16,), 0)

    runmax = [jnp.full((16,), _NEG, jnp.float32) for _ in range(8)]
    runidx = [jnp.zeros((16,), jnp.int32) for _ in range(8)]

    bufs = (buf0, buf1)
    sems = (sem0, sem1)

    def cstart(jj):
        return pl.multiple_of((2 * jj + cid) * _CHW, _CHW)

    copies = [None] * _NCHW
    copies[0] = pltpu.async_copy(
        out_hbm.at[pl.ds(row0, 8), pl.ds(cstart(0), _CHW)], bufs[0], sems[0])
    for jj in range(_NCHW):
        if jj + 1 < _NCHW:
            copies[jj + 1] = pltpu.async_copy(
                out_hbm.at[pl.ds(row0, 8), pl.ds(cstart(jj + 1), _CHW)],
                bufs[(jj + 1) % 2], sems[(jj + 1) % 2])
        copies[jj].wait()
        buf = bufs[jj % 2]
        pos0 = cstart(jj) + lane

        def body(i, carry, buf=buf):
            st = list(carry[:-1])
            pos = carry[-1]
            off = i * 128
            for k in range(8):
                p = pos + k * 16
                for r in range(8):
                    rm, ri = _update(buf[r, pl.ds(off + k * 16, 16)], p,
                                     st[2 * r], st[2 * r + 1])
                    st[2 * r], st[2 * r + 1] = rm, ri
            return tuple(st) + (pos + 128,)

        carry = []
        for r in range(8):
            carry += [runmax[r], runidx[r]]
        carry = lax.fori_loop(0, _CHT, body, tuple(carry) + (pos0,))
        for r in range(8):
            runmax[r], runidx[r] = carry[2 * r], carry[2 * r + 1]

    pltpu.sync_copy(out_hbm.at[pl.ds(row0, 8), pl.ds(_TAILC, _TAILW)], tbuf)
    for k in range(_TAILW // 16):
        p = _TAILC + k * 16 + lane
        for r in range(8):
            runmax[r], runidx[r] = _update(
                tbuf[r, pl.ds(k * 16, 16)], p, runmax[r], runidx[r])

    maxv_vec = jnp.full((16,), _NEG, jnp.float32)
    idx_vec = jnp.zeros((16,), jnp.int32)
    for r in range(8):
        val, idx = runmax[r], runidx[r]
        for s in (1, 2, 4, 8):
            perm = lane ^ s
            oval = _shuffle(val, perm)
            oidx = _shuffle(idx, perm)
            take = (oval > val) | ((oval == val) & (oidx < idx))
            val = jnp.where(take, oval, val)
            idx = jnp.where(take, oidx, idx)
        maxv_vec = jnp.where(lane == r, val, maxv_vec)
        idx_vec = jnp.where(lane == r, idx, idx_vec)

    cin = idx_vec & 127
    cseg = cin >> 4
    clane = cin & 15
    lsel_vec = jnp.zeros((16,), jnp.float32)
    for r in range(8):
        tile = pl.multiple_of((idx_vec[r] >> 7) << 7, 128)
        pltpu.async_copy(
            logit_hbm.at[pl.ds(row0, 8), pl.ds(tile, 128)], gbuf, gsem).wait()
        got = jnp.zeros((16,), jnp.float32)
        for seg in range(8):
            v = gbuf[r, pl.ds(seg * 16, 16)]
            got = jnp.where(cseg == seg, _shuffle(v, clane), got)
        lsel_vec = jnp.where(lane == r, got, lsel_vec)

    valv[...] = maxv_vec
    idxv[...] = idx_vec
    lselv[...] = lsel_vec
    pltpu.sync_copy(valv, val_hbm.at[pl.ds(wid * 16, 16)])
    pltpu.sync_copy(idxv, idx_hbm.at[pl.ds(wid * 16, 16)])
    pltpu.sync_copy(lselv, lsel_hbm.at[pl.ds(wid * 16, 16)])


_sc_argmax_gather = functools.partial(
    pl.kernel,
    out_type=[
        jax.ShapeDtypeStruct((512,), jnp.float32),
        jax.ShapeDtypeStruct((512,), jnp.int32),
        jax.ShapeDtypeStruct((512,), jnp.float32),
    ],
    mesh=plsc.VectorSubcoreMesh(core_axis_name="c", subcore_axis_name="s"),
    scratch_types=[
        pltpu.VMEM((8, _CHW), jnp.float32),
        pltpu.VMEM((8, _CHW), jnp.float32),
        pltpu.VMEM((8, _TAILW), jnp.float32),
        pltpu.VMEM((8, 128), jnp.float32),
        pltpu.VMEM((16,), jnp.float32),
        pltpu.VMEM((16,), jnp.int32),
        pltpu.VMEM((16,), jnp.float32),
        pltpu.SemaphoreType.DMA,
        pltpu.SemaphoreType.DMA,
        pltpu.SemaphoreType.DMA,
    ],
)(_sc_body)


def _tc_sums_body(l_ref, ent_ref, logs_ref):
    l = l_ref[...]
    s = jnp.sum(l, axis=1, keepdims=True)
    sll = jnp.sum(l * jnp.log(l), axis=1, keepdims=True)
    logs = jnp.log(s)
    ent_ref[...] = jnp.broadcast_to(logs - sll / s, (_BR, 128))
    logs_ref[...] = jnp.broadcast_to(logs, (_BR, 128))


def _combine_body(v0_ref, v1_ref, i0_ref, i1_ref, s0_ref, s1_ref, logs_ref,
                  alp_ref):
    v0, v1 = v0_ref[...], v1_ref[...]
    i0, i1 = i0_ref[...], i1_ref[...]
    take1 = (v1 > v0) | ((v1 == v0) & (i1 < i0))
    lsel = jnp.where(take1, s1_ref[...], s0_ref[...])
    alp_ref[...] = jnp.log(lsel) - logs_ref[...]


def kernel(logits, outputs):
    val, idx, lsel = _sc_argmax_gather(outputs, logits)
    ent, logs = pl.pallas_call(
        _tc_sums_body,
        grid=(_NBLK,),
        in_specs=[pl.BlockSpec((_BR, _C), lambda i: (i, 0))],
        out_specs=[
            pl.BlockSpec((_BR, 128), lambda i: (i, 0)),
            pl.BlockSpec((_BR, 128), lambda i: (i, 0)),
        ],
        out_shape=[
            jax.ShapeDtypeStruct((_R, 128), jnp.float32),
            jax.ShapeDtypeStruct((_R, 128), jnp.float32),
        ],
    )(logits)
    val2 = val.reshape(2, 16, 16)[:, :, :8].reshape(2, _R)
    idx2 = idx.reshape(2, 16, 16)[:, :, :8].reshape(2, _R)
    lsel2 = lsel.reshape(2, 16, 16)[:, :, :8].reshape(2, _R)
    logs_row = logs[:, 0].reshape(1, _R)
    alp = pl.pallas_call(
        _combine_body,
        out_shape=jax.ShapeDtypeStruct((1, _R), jnp.float32),
    )(val2[0:1], val2[1:2], idx2[0:1], idx2[1:2],
      lsel2[0:1], lsel2[1:2], logs_row)
    return (alp.reshape(_R), ent[:, 0])


# speedup vs baseline: 5.3845x; 2.9345x over previous
import functools

import jax
import jax.numpy as jnp
from jax import lax
from jax.experimental import pallas as pl
from jax.experimental.pallas import tpu as pltpu
from jax.experimental.pallas import tpu_sc as plsc

_R, _C = 128, 100000
_NW = 32
_CPW = 3120
_MAIN = _NW * _CPW
_TAILW = _C - _MAIN
_CHW = 120
_NCH = _CPW // _CHW
_NEG = -3.4e38
_BIG = 2**30

def _sc_body(tout_hbm, val_hbm, idx_hbm, buf0, buf1, tbuf, valv, idxv,
             sem0, sem1):
    cid = lax.axis_index("c")
    sid = lax.axis_index("s")
    wid = cid * 16 + sid
    col0 = wid * _CPW

    rm = [jnp.full((16,), _NEG, jnp.float32) for _ in range(8)]
    ri = [jnp.zeros((16,), jnp.int32) for _ in range(8)]

    bufs = (buf0, buf1)
    sems = (sem0, sem1)

    def cstart(j):
        return pl.multiple_of(col0 + j * _CHW, 8)

    copies = [None] * _NCH
    copies[0] = pltpu.async_copy(
        tout_hbm.at[pl.ds(cstart(0), _CHW)], bufs[0], sems[0])
    for j in range(_NCH):
        if j + 1 < _NCH:
            copies[j + 1] = pltpu.async_copy(
                tout_hbm.at[pl.ds(cstart(j + 1), _CHW)],
                bufs[(j + 1) % 2], sems[(j + 1) % 2])
        copies[j].wait()
        buf = bufs[j % 2]
        pos0 = jnp.full((16,), 0, jnp.int32) + (col0 + j * _CHW)

        def body(i, carry, buf=buf):
            st = list(carry[:-1])
            pos = carry[-1]
            for u in range(2):
                c = i * 2 + u
                p = pos + u
                for g in range(8):
                    v = buf[c, pl.ds(g * 16, 16)]
                    upd = v > st[2 * g]
                    st[2 * g] = jnp.where(upd, v, st[2 * g])
                    st[2 * g + 1] = jnp.where(upd, p, st[2 * g + 1])
            return tuple(st) + (pos + 2,)

        carry = []
        for g in range(8):
            carry += [rm[g], ri[g]]
        carry = lax.fori_loop(0, _CHW // 2, body, tuple(carry) + (pos0,))
        for g in range(8):
            rm[g], ri[g] = carry[2 * g], carry[2 * g + 1]

    pltpu.sync_copy(tout_hbm.at[pl.ds(_MAIN, _TAILW)], tbuf)
    for c in range(_TAILW):
        p = jnp.full((16,), _MAIN + c, jnp.int32)
        for g in range(8):
            v = tbuf[c, pl.ds(g * 16, 16)]
            upd = v > rm[g]
            rm[g] = jnp.where(upd, v, rm[g])
            ri[g] = jnp.where(upd, p, ri[g])

    for g in range(8):
        valv[pl.ds(g * 16, 16)] = rm[g]
        idxv[pl.ds(g * 16, 16)] = ri[g]
    pltpu.sync_copy(valv, val_hbm.at[pl.ds(wid * 128, 128)])
    pltpu.sync_copy(idxv, idx_hbm.at[pl.ds(wid * 128, 128)])


_sc_argmax = functools.partial(
    pl.kernel,
    out_type=[
        jax.ShapeDtypeStruct((_NW * 128,), jnp.float32),
        jax.ShapeDtypeStruct((_NW * 128,), jnp.int32),
    ],
    mesh=plsc.VectorSubcoreMesh(core_axis_name="c", subcore_axis_name="s"),
    scratch_types=[
        pltpu.VMEM((_CHW, 128), jnp.float32),
        pltpu.VMEM((_CHW, 128), jnp.float32),
        pltpu.VMEM((_TAILW, 128), jnp.float32),
        pltpu.VMEM((128,), jnp.float32),
        pltpu.VMEM((128,), jnp.int32),
        pltpu.SemaphoreType.DMA,
        pltpu.SemaphoreType.DMA,
    ],
)(_sc_body)


_SBLK = 10000


def _tc_sums_body(l_ref, s_ref, sll_ref):
    l = l_ref[...]
    ps = jnp.sum(l, axis=0, keepdims=True)
    psll = jnp.sum(l * jnp.log(l), axis=0, keepdims=True)

    @pl.when(pl.program_id(0) == 0)
    def _():
        s_ref[...] = jnp.zeros_like(s_ref)
        sll_ref[...] = jnp.zeros_like(sll_ref)

    s_ref[...] += ps
    sll_ref[...] += psll


def _merge_body(val_ref, idx_ref, s_ref, sll_ref, ri_ref, ent_ref, logs_ref):
    val = val_ref[...]
    idx = idx_ref[...]
    m = jnp.max(val, axis=0, keepdims=True)
    ri = jnp.min(jnp.where(val == m, idx, _BIG), axis=0, keepdims=True)
    s = s_ref[...]
    logs = jnp.log(s)
    ri_ref[...] = ri
    ent_ref[...] = logs - sll_ref[...] / s
    logs_ref[...] = logs


def _gather_body(ri_smem, ri_vmem, tl_hbm, logs_ref, alp_ref, gbuf, sem):
    copies = []
    for r in range(_R):
        base = pl.multiple_of((ri_smem[0, r] >> 3) << 3, 8)
        copies.append(pltpu.make_async_copy(
            tl_hbm.at[pl.ds(base, 8)], gbuf.at[r], sem))
        copies[-1].start()
    for c in copies:
        c.wait()
    g = gbuf[...]
    sub = ri_vmem[...] & 7
    row_i = lax.broadcasted_iota(jnp.int32, (_R, 8, 128), 0)
    sub_i = lax.broadcasted_iota(jnp.int32, (_R, 8, 128), 1)
    lane_i = lax.broadcasted_iota(jnp.int32, (_R, 8, 128), 2)
    pick = (lane_i == row_i) & (sub_i == sub.reshape(_R)[:, None, None])
    lsel = jnp.sum(jnp.where(pick, g, 0.0), axis=(1, 2))
    alp_ref[...] = jnp.log(lsel).reshape(1, _R) - logs_ref[...]


def kernel(logits, outputs):
    tl = logits.T
    tout = outputs.T
    val, idx = _sc_argmax(tout)
    s, sll = pl.pallas_call(
        _tc_sums_body,
        grid=(_C // _SBLK,),
        in_specs=[pl.BlockSpec((_SBLK, 128), lambda i: (i, 0))],
        out_specs=[
            pl.BlockSpec((1, 128), lambda i: (0, 0)),
            pl.BlockSpec((1, 128), lambda i: (0, 0)),
        ],
        out_shape=[
            jax.ShapeDtypeStruct((1, 128), jnp.float32),
            jax.ShapeDtypeStruct((1, 128), jnp.float32),
        ],
    )(tl)
    ri, ent, logs = pl.pallas_call(
        _merge_body,
        out_shape=[
            jax.ShapeDtypeStruct((1, 128), jnp.int32),
            jax.ShapeDtypeStruct((1, 128), jnp.float32),
            jax.ShapeDtypeStruct((1, 128), jnp.float32),
        ],
    )(val.reshape(_NW, 128), idx.reshape(_NW, 128), s, sll)
    alp = pl.pallas_call(
        _gather_body,
        in_specs=[
            pl.BlockSpec(memory_space=pltpu.SMEM),
            pl.BlockSpec(memory_space=pltpu.VMEM),
            pl.BlockSpec(memory_space=pltpu.MemorySpace.HBM),
            pl.BlockSpec(memory_space=pltpu.VMEM),
        ],
        out_specs=pl.BlockSpec(memory_space=pltpu.VMEM),
        out_shape=jax.ShapeDtypeStruct((1, _R), jnp.float32),
        scratch_shapes=[
            pltpu.VMEM((_R, 8, 128), jnp.float32),
            pltpu.SemaphoreType.DMA,
        ],
    )(ri, ri, tl, logs)
    return (alp.reshape(_R), ent.reshape(_R))


# speedup vs baseline: 5.5293x; 1.0269x over previous
import functools

import jax
import jax.numpy as jnp
from jax import lax
from jax.experimental import pallas as pl
from jax.experimental.pallas import tpu as pltpu
from jax.experimental.pallas import tpu_sc as plsc

_R, _C = 128, 100000
_NW = 32
_CPW = 3120
_MAIN = _NW * _CPW
_TAILW = _C - _MAIN
_CHW = 120
_NCH = _CPW // _CHW
_NEG = -3.4e38
_BIG = 2**30

def _sc_body(tout_hbm, val_hbm, idx_hbm, buf0, buf1, tbuf, valv, idxv,
             sem0, sem1, tsem):
    cid = lax.axis_index("c")
    sid = lax.axis_index("s")
    wid = cid * 16 + sid
    col0 = wid * _CPW

    bufs = (buf0, buf1)
    sems = (sem0, sem1)

    def cstart(j):
        return pl.multiple_of(col0 + j * _CHW, 8)

    def scan_cols(buf, pos0, carry, ncols):
        def body(i, c):
            st = list(c[:-1])
            pos = c[-1]
            for u in range(2):
                col = i * 2 + u
                p = pos + u
                for g in range(8):
                    v = buf[col, pl.ds(g * 16, 16)]
                    upd = v > st[2 * g]
                    st[2 * g] = jnp.where(upd, v, st[2 * g])
                    st[2 * g + 1] = jnp.where(upd, p, st[2 * g + 1])
            return tuple(st) + (pos + 2,)

        out = lax.fori_loop(0, ncols // 2, body, tuple(carry) + (pos0,))
        return out[:-1]

    pltpu.async_copy(tout_hbm.at[pl.ds(_MAIN, _TAILW)], tbuf, tsem)
    pltpu.async_copy(tout_hbm.at[pl.ds(cstart(0), _CHW)], bufs[0], sems[0])
    pltpu.async_copy(tout_hbm.at[pl.ds(cstart(1), _CHW)], bufs[1], sems[1])

    carry = []
    for g in range(8):
        carry += [jnp.full((16,), _NEG, jnp.float32),
                  jnp.zeros((16,), jnp.int32)]
    carry = tuple(carry)

    def outer(jp, carry):
        for b in range(2):
            j = jp * 2 + b
            pltpu.make_async_copy(
                tout_hbm.at[pl.ds(cstart(0), _CHW)], bufs[b],
                sems[b]).wait()
            c2 = scan_cols(bufs[b], jnp.full((16,), 0, jnp.int32)
                           + (col0 + j * _CHW), carry, _CHW)

            @pl.when(j + 2 < _NCH)
            def _():
                pltpu.async_copy(
                    tout_hbm.at[pl.ds(cstart(j + 2), _CHW)], bufs[b],
                    sems[b])

            carry = c2
        return carry

    carry = lax.fori_loop(0, _NCH // 2, outer, carry)

    pltpu.make_async_copy(
        tout_hbm.at[pl.ds(_MAIN, _TAILW)], tbuf, tsem).wait()
    carry = scan_cols(tbuf, jnp.full((16,), _MAIN, jnp.int32), carry, _TAILW)

    for g in range(8):
        valv[pl.ds(g * 16, 16)] = carry[2 * g]
        idxv[pl.ds(g * 16, 16)] = carry[2 * g + 1]
    pltpu.sync_copy(valv, val_hbm.at[pl.ds(wid * 128, 128)])
    pltpu.sync_copy(idxv, idx_hbm.at[pl.ds(wid * 128, 128)])


_sc_argmax = functools.partial(
    pl.kernel,
    out_type=[
        jax.ShapeDtypeStruct((_NW * 128,), jnp.float32),
        jax.ShapeDtypeStruct((_NW * 128,), jnp.int32),
    ],
    mesh=plsc.VectorSubcoreMesh(core_axis_name="c", subcore_axis_name="s"),
    scratch_types=[
        pltpu.VMEM((_CHW, 128), jnp.float32),
        pltpu.VMEM((_CHW, 128), jnp.float32),
        pltpu.VMEM((_TAILW, 128), jnp.float32),
        pltpu.VMEM((128,), jnp.float32),
        pltpu.VMEM((128,), jnp.int32),
        pltpu.SemaphoreType.DMA,
        pltpu.SemaphoreType.DMA,
        pltpu.SemaphoreType.DMA,
    ],
)(_sc_body)


_SBLK = 10000


def _tc_sums_body(l_ref, s_ref, sll_ref):
    l = l_ref[...]
    ps = jnp.sum(l, axis=0, keepdims=True)
    psll = jnp.sum(l * jnp.log(l), axis=0, keepdims=True)

    @pl.when(pl.program_id(0) == 0)
    def _():
        s_ref[...] = jnp.zeros_like(s_ref)
        sll_ref[...] = jnp.zeros_like(sll_ref)

    s_ref[...] += ps
    sll_ref[...] += psll


def _merge_body(val_ref, idx_ref, s_ref, sll_ref, ri_ref, ent_ref, logs_ref):
    val = val_ref[...]
    idx = idx_ref[...]
    m = jnp.max(val, axis=0, keepdims=True)
    ri = jnp.min(jnp.where(val == m, idx, _BIG), axis=0, keepdims=True)
    s = s_ref[...]
    logs = jnp.log(s)
    ri_ref[...] = ri
    ent_ref[...] = logs - sll_ref[...] / s
    logs_ref[...] = logs


def _gather_body(ri_smem, ri_vmem, tl_hbm, logs_ref, alp_ref, gbuf, sem):
    copies = []
    for r in range(_R):
        base = pl.multiple_of((ri_smem[0, r] >> 3) << 3, 8)
        copies.append(pltpu.make_async_copy(
            tl_hbm.at[pl.ds(base, 8)], gbuf.at[r], sem))
        copies[-1].start()
    for c in copies:
        c.wait()
    g = gbuf[...]
    sub = ri_vmem[...] & 7
    row_i = lax.broadcasted_iota(jnp.int32, (_R, 8, 128), 0)
    sub_i = lax.broadcasted_iota(jnp.int32, (_R, 8, 128), 1)
    lane_i = lax.broadcasted_iota(jnp.int32, (_R, 8, 128), 2)
    pick = (lane_i == row_i) & (sub_i == sub.reshape(_R)[:, None, None])
    lsel = jnp.sum(jnp.where(pick, g, 0.0), axis=(1, 2))
    alp_ref[...] = jnp.log(lsel).reshape(1, _R) - logs_ref[...]


def kernel(logits, outputs):
    tl = logits.T
    tout = outputs.T
    val, idx = _sc_argmax(tout)
    s, sll = pl.pallas_call(
        _tc_sums_body,
        grid=(_C // _SBLK,),
        in_specs=[pl.BlockSpec((_SBLK, 128), lambda i: (i, 0))],
        out_specs=[
            pl.BlockSpec((1, 128), lambda i: (0, 0)),
            pl.BlockSpec((1, 128), lambda i: (0, 0)),
        ],
        out_shape=[
            jax.ShapeDtypeStruct((1, 128), jnp.float32),
            jax.ShapeDtypeStruct((1, 128), jnp.float32),
        ],
    )(tl)
    ri, ent, logs = pl.pallas_call(
        _merge_body,
        out_shape=[
            jax.ShapeDtypeStruct((1, 128), jnp.int32),
            jax.ShapeDtypeStruct((1, 128), jnp.float32),
            jax.ShapeDtypeStruct((1, 128), jnp.float32),
        ],
    )(val.reshape(_NW, 128), idx.reshape(_NW, 128), s, sll)
    alp = pl.pallas_call(
        _gather_body,
        in_specs=[
            pl.BlockSpec(memory_space=pltpu.SMEM),
            pl.BlockSpec(memory_space=pltpu.VMEM),
            pl.BlockSpec(memory_space=pltpu.MemorySpace.HBM),
            pl.BlockSpec(memory_space=pltpu.VMEM),
        ],
        out_specs=pl.BlockSpec(memory_space=pltpu.VMEM),
        out_shape=jax.ShapeDtypeStruct((1, _R), jnp.float32),
        scratch_shapes=[
            pltpu.VMEM((_R, 8, 128), jnp.float32),
            pltpu.SemaphoreType.DMA,
        ],
    )(ri, ri, tl, logs)
    return (alp.reshape(_R), ent.reshape(_R))
